# Initial kernel scaffold; baseline (speedup 1.0000x reference)
import jax
import jax.numpy as jnp
from jax.experimental import pallas as pl


def kernel(x, edge_index, W0, b0, g0, be0, W1, b1, g1, be1, Wg, att_src, att_dst, bg):
    raise NotImplementedError("write your pallas kernel here")


# speedup vs baseline: 9.7244x; 9.7244x over previous
import functools

import jax
import jax.numpy as jnp
from jax import lax
from jax.experimental import pallas as pl
from jax.experimental.pallas import tpu as pltpu
from jax.experimental.pallas import tpu_sc as plsc

N = 10000
E = 320000
D = 128
H = 4

NC = 2
NS = 16
NW = NC * NS
EPW = E // NW
N2 = 10112
RPTW = N2 // NS
K = 80
NCHUNK = EPW // K

_mesh = plsc.VectorSubcoreMesh(
    core_axis_name="c", subcore_axis_name="s", num_cores=NC, num_subcores=NS)
_sc_params = pltpu.CompilerParams(
    use_tc_tiling_on_sc=False, needs_layout_passes=False)


def _fill_zero_2d(ref, nrows, ncols):
  zero16 = jnp.zeros((16,), jnp.float32)
  cpr = ncols // 16

  def body(t, _):
    i = t // cpr
    c = t % cpr
    ref[i, pl.ds(c * 16, 16)] = zero16
    return 0

  lax.fori_loop(0, nrows * cpr, body, 0)


def _edge_base(cid, sid):
  return (cid * NS + sid) * EPW


@functools.partial(
    pl.kernel,
    out_type=jax.ShapeDtypeStruct((NC * N2, 16), jnp.float32),
    mesh=_mesh,
    compiler_params=_sc_params,
    scratch_types=[
        pltpu.VMEM((K,), jnp.int32),
        pltpu.VMEM((K, 16), jnp.float32),
        pltpu.VMEM((RPTW, 16), jnp.float32),
        pltpu.VMEM_SHARED((N2, 16), jnp.float32),
    ],
)
def _sc_deg(dst_hbm, out_hbm, didx, onesv, zb, acc_sh):
  cid = lax.axis_index("c")
  sid = lax.axis_index("s")
  base = _edge_base(cid, sid)

  _fill_zero_2d(zb, RPTW, 16)
  pltpu.sync_copy(zb, acc_sh.at[pl.ds(sid * RPTW, RPTW)])

  one_row = (lax.iota(jnp.int32, 16) == 0).astype(jnp.float32)

  def fillones(i, _):
    onesv[i, pl.ds(0, 16)] = one_row
    return 0

  lax.fori_loop(0, K, fillones, 0)
  plsc.subcore_barrier()

  def chunk(i, _):
    pltpu.sync_copy(dst_hbm.at[pl.ds(base + i * K, K)], didx)
    pltpu.sync_copy(onesv, acc_sh.at[didx], add=True)
    return 0

  lax.fori_loop(0, NCHUNK, chunk, 0)
  plsc.subcore_barrier()
  pltpu.sync_copy(acc_sh.at[pl.ds(sid * RPTW, RPTW)],
                  out_hbm.at[pl.ds(cid * N2 + sid * RPTW, RPTW)])


@functools.partial(
    pl.kernel,
    out_type=jax.ShapeDtypeStruct((2 * NC * N2, D // 2), jnp.float32),
    mesh=_mesh,
    compiler_params=_sc_params,
    scratch_types=[
        pltpu.VMEM((K,), jnp.int32),
        pltpu.VMEM((K,), jnp.int32),
        pltpu.VMEM((K, D // 2), jnp.float32),
        pltpu.VMEM((120, D // 2), jnp.float32),
        pltpu.VMEM_SHARED((N2, D // 2), jnp.float32),
    ],
)
def _sc_gcn(hsa_hbm, hsb_hbm, src_hbm, dst_hbm, out_hbm,
            sidx, didx, rows, zb, acc_sh):
  cid = lax.axis_index("c")
  sid = lax.axis_index("s")
  base = _edge_base(cid, sid)
  hcols = (hsa_hbm, hsb_hbm)
  _fill_zero_2d(zb, 120, D // 2)

  for c in range(2):
    for j in range(5):
      pltpu.sync_copy(zb, acc_sh.at[pl.ds(sid * RPTW + j * 120, 120)])
    pltpu.sync_copy(zb.at[pl.ds(0, 32)],
                    acc_sh.at[pl.ds(sid * RPTW + 600, 32)])
    plsc.subcore_barrier()

    def chunk(i, _, c=c):
      pltpu.sync_copy(src_hbm.at[pl.ds(base + i * K, K)], sidx)
      pltpu.sync_copy(hcols[c].at[sidx], rows)
      pltpu.sync_copy(dst_hbm.at[pl.ds(base + i * K, K)], didx)
      pltpu.sync_copy(rows, acc_sh.at[didx], add=True)
      return 0

    lax.fori_loop(0, NCHUNK, chunk, 0)
    plsc.subcore_barrier()
    pltpu.sync_copy(
        acc_sh.at[pl.ds(sid * RPTW, RPTW)],
        out_hbm.at[pl.ds(c * NC * N2 + cid * N2 + sid * RPTW, RPTW)])


@functools.partial(
    pl.kernel,
    out_type=jax.ShapeDtypeStruct((NC * N2, 16), jnp.float32),
    mesh=_mesh,
    compiler_params=_sc_params,
    scratch_types=[
        pltpu.VMEM((K,), jnp.int32),
        pltpu.VMEM((K,), jnp.int32),
        pltpu.VMEM((K, 16), jnp.float32),
        pltpu.VMEM((H * N,), jnp.float32),
        pltpu.VMEM((H * N,), jnp.float32),
        pltpu.VMEM((RPTW, 16), jnp.float32),
        pltpu.VMEM_SHARED((N2, 16), jnp.float32),
    ],
)
def _sc_den(src_hbm, dst_hbm, est_hbm, edt_hbm, out_hbm,
            sidx, didx, eebuf, est_v, edt_v, zb, acc_sh):
  cid = lax.axis_index("c")
  sid = lax.axis_index("s")
  base = _edge_base(cid, sid)

  _fill_zero_2d(zb, RPTW, 16)
  pltpu.sync_copy(zb, acc_sh.at[pl.ds(sid * RPTW, RPTW)])
  _fill_zero_2d(eebuf, K, 16)
  pltpu.sync_copy(est_hbm, est_v)
  pltpu.sync_copy(edt_hbm, edt_v)
  plsc.subcore_barrier()

  hconst = [jnp.full((16,), h, jnp.int32) for h in range(H)]

  def chunk(i, _):
    pltpu.sync_copy(src_hbm.at[pl.ds(base + i * K, K)], sidx)
    pltpu.sync_copy(dst_hbm.at[pl.ds(base + i * K, K)], didx)
    for g in range(K // 16):
      s16 = sidx[pl.ds(g * 16, 16)]
      d16 = didx[pl.ds(g * 16, 16)]
      rowsid = jnp.arange(g * 16, g * 16 + 16, dtype=jnp.int32)
      for h in range(H):
        a = plsc.load_gather(est_v, [s16 + h * N])
        b = plsc.load_gather(edt_v, [d16 + h * N])
        s = a + b
        e = jnp.where(s >= 0, s, 0.2 * s)
        ee = jnp.exp(e)
        plsc.store_scatter(eebuf, [rowsid, hconst[h]], ee)
    pltpu.sync_copy(eebuf, acc_sh.at[didx], add=True)
    return 0

  lax.fori_loop(0, NCHUNK, chunk, 0)
  plsc.subcore_barrier()
  pltpu.sync_copy(acc_sh.at[pl.ds(sid * RPTW, RPTW)],
                  out_hbm.at[pl.ds(cid * N2 + sid * RPTW, RPTW)])


@functools.partial(
    pl.kernel,
    out_type=jax.ShapeDtypeStruct((2 * NC * N2, D // 2), jnp.float32),
    mesh=_mesh,
    compiler_params=_sc_params,
    scratch_types=[
        pltpu.VMEM((K,), jnp.int32),
        pltpu.VMEM((K,), jnp.int32),
        pltpu.VMEM((K, D // 2), jnp.float32),
        pltpu.VMEM((K,), jnp.float32),
        pltpu.VMEM((N,), jnp.float32),
        pltpu.VMEM((N,), jnp.float32),
        pltpu.VMEM((N,), jnp.float32),
        pltpu.VMEM((120, D // 2), jnp.float32),
        pltpu.VMEM_SHARED((N2, D // 2), jnp.float32),
    ],
)
def _sc_gat(src_hbm, dst_hbm, est_hbm, edt_hbm, rdent_hbm,
            hga0, hga1, hga2, hga3, hgb0, hgb1, hgb2, hgb3, out_hbm,
            sidx, didx, rows, abuf, esh_v, edh_v, rdh_v, zb, acc_sh):
  cid = lax.axis_index("c")
  sid = lax.axis_index("s")
  base = _edge_base(cid, sid)
  hgs = ((hga0, hga1, hga2, hga3), (hgb0, hgb1, hgb2, hgb3))
  _fill_zero_2d(zb, 120, D // 2)

  for c in range(2):
    for j in range(5):
      pltpu.sync_copy(zb, acc_sh.at[pl.ds(sid * RPTW + j * 120, 120)])
    pltpu.sync_copy(zb.at[pl.ds(0, 32)],
                    acc_sh.at[pl.ds(sid * RPTW + 600, 32)])
    plsc.subcore_barrier()

    for h in range(H):
      pltpu.sync_copy(est_hbm.at[pl.ds(h * N, N)], esh_v)
      pltpu.sync_copy(edt_hbm.at[pl.ds(h * N, N)], edh_v)
      pltpu.sync_copy(rdent_hbm.at[pl.ds(h * N, N)], rdh_v)

      def chunk(i, _, c=c, h=h):
        pltpu.sync_copy(src_hbm.at[pl.ds(base + i * K, K)], sidx)
        pltpu.sync_copy(dst_hbm.at[pl.ds(base + i * K, K)], didx)
        pltpu.sync_copy(hgs[c][h].at[sidx], rows)
        for g in range(K // 16):
          s16 = sidx[pl.ds(g * 16, 16)]
          d16 = didx[pl.ds(g * 16, 16)]
          a = plsc.load_gather(esh_v, [s16])
          b = plsc.load_gather(edh_v, [d16])
          s = a + b
          e = jnp.where(s >= 0, s, 0.2 * s)
          ee = jnp.exp(e)
          r = plsc.load_gather(rdh_v, [d16])
          abuf[pl.ds(g * 16, 16)] = ee * r

        def edge_body(k, _):
          k16 = jnp.full((16,), 0, jnp.int32) + k
          spl = plsc.load_gather(abuf, [k16])
          for cc in range(D // 32):
            cidx = jnp.arange(cc * 16, cc * 16 + 16, dtype=jnp.int32)
            v = plsc.load_gather(rows, [k16, cidx])
            plsc.store_scatter(rows, [k16, cidx], v * spl)
          return 0

        lax.fori_loop(0, K, edge_body, 0)
        pltpu.sync_copy(rows, acc_sh.at[didx], add=True)
        return 0

      lax.fori_loop(0, NCHUNK, chunk, 0)

    plsc.subcore_barrier()
    pltpu.sync_copy(
        acc_sh.at[pl.ds(sid * RPTW, RPTW)],
        out_hbm.at[pl.ds(c * NC * N2 + cid * N2 + sid * RPTW, RPTW)])


RB = 2000
NG = N // RB

_row_spec = lambda cols: pl.BlockSpec((RB, cols), lambda i: (i, 0))
_full_spec = lambda r, c: pl.BlockSpec((r, c), lambda i: (0, 0))


def _dinv_of(dp0, dp1):
  return 1.0 / jnp.sqrt(dp0 + dp1 + 1.0)


def _tc1_body(x_ref, w_ref, dp0_ref, dp1_ref, hs_ref):
  dinv = _dinv_of(dp0_ref[...], dp1_ref[...])
  h = jnp.dot(x_ref[...], w_ref[...], preferred_element_type=jnp.float32, precision=lax.Precision.HIGHEST)
  hs_ref[...] = h * dinv


def _tc1(x, w, dp0, dp1):
  return pl.pallas_call(
      _tc1_body,
      grid=(NG,),
      in_specs=[_row_spec(D), _full_spec(D, D), _row_spec(1), _row_spec(1)],
      out_specs=_row_spec(D),
      out_shape=jax.ShapeDtypeStruct((N, D), jnp.float32),
  )(x, w, dp0, dp1)


def _ln_relu(t, g, be):
  m = jnp.mean(t, axis=-1, keepdims=True)
  v = jnp.mean((t - m) * (t - m), axis=-1, keepdims=True)
  t = (t - m) / jnp.sqrt(v + 1e-5) * g + be
  return jnp.maximum(t, 0.0)


def _tc2_body(p0_ref, p1_ref, hs_ref, dp0_ref, dp1_ref, b_ref, g_ref,
              be_ref, w_ref, out_ref):
  dinv = _dinv_of(dp0_ref[...], dp1_ref[...])
  t = dinv * (p0_ref[...] + p1_ref[...] + hs_ref[...]) + b_ref[...]
  t = _ln_relu(t, g_ref[...], be_ref[...])
  out_ref[...] = jnp.dot(
      t, w_ref[...], preferred_element_type=jnp.float32, precision=lax.Precision.HIGHEST) * dinv


def _tc2(p0, p1, hs, dp0, dp1, b, g, be, w):
  return pl.pallas_call(
      _tc2_body,
      grid=(NG,),
      in_specs=[_row_spec(D), _row_spec(D), _row_spec(D), _row_spec(1),
                _row_spec(1), _full_spec(1, D), _full_spec(1, D),
                _full_spec(1, D), _full_spec(D, D)],
      out_specs=_row_spec(D),
      out_shape=jax.ShapeDtypeStruct((N, D), jnp.float32),
  )(p0, p1, hs, dp0, dp1, b, g, be, w)


def _tc3_body(p0_ref, p1_ref, hs_ref, dp0_ref, dp1_ref, b_ref, g_ref,
              be_ref, wg_ref, asrc_ref, adst_ref,
              hg0_ref, hg1_ref, hg2_ref, hg3_ref, es_ref, ed_ref):
  dinv = _dinv_of(dp0_ref[...], dp1_ref[...])
  t = dinv * (p0_ref[...] + p1_ref[...] + hs_ref[...]) + b_ref[...]
  t = _ln_relu(t, g_ref[...], be_ref[...])
  hg = jnp.dot(t, wg_ref[...], preferred_element_type=jnp.float32, precision=lax.Precision.HIGHEST)
  hg_refs = (hg0_ref, hg1_ref, hg2_ref, hg3_ref)
  for h in range(H):
    hg_refs[h][...] = hg[:, h * D:(h + 1) * D]
  es = jnp.dot(hg, asrc_ref[...], preferred_element_type=jnp.float32, precision=lax.Precision.HIGHEST)
  ed = jnp.dot(hg, adst_ref[...], preferred_element_type=jnp.float32, precision=lax.Precision.HIGHEST)
  es_ref[...] = es
  ed_ref[...] = ed


def _tc3(p0, p1, hs, dp0, dp1, b, g, be, wg, a_src_mat, a_dst_mat):
  return pl.pallas_call(
      _tc3_body,
      grid=(NG,),
      in_specs=[_row_spec(D), _row_spec(D), _row_spec(D), _row_spec(1),
                _row_spec(1), _full_spec(1, D), _full_spec(1, D),
                _full_spec(1, D), _full_spec(D, H * D),
                _full_spec(H * D, H), _full_spec(H * D, H)],
      out_specs=[_row_spec(D), _row_spec(D), _row_spec(D), _row_spec(D),
                 _row_spec(H), _row_spec(H)],
      out_shape=[jax.ShapeDtypeStruct((N, D), jnp.float32)] * 4 + [
          jax.ShapeDtypeStruct((N, H), jnp.float32),
          jax.ShapeDtypeStruct((N, H), jnp.float32),
      ],
  )(p0, p1, hs, dp0, dp1, b, g, be, wg, a_src_mat, a_dst_mat)


def _tc4_body(dn0_ref, dn1_ref, es_ref, ed_ref, rden_ref, ees_ref):
  s = es_ref[...] + ed_ref[...]
  e = jnp.where(s >= 0, s, 0.2 * s)
  ee = jnp.exp(e)
  den = dn0_ref[...] + dn1_ref[...] + ee
  rden_ref[...] = 0.25 / (den + 1e-30)
  ees_ref[...] = ee


def _tc4(dn0, dn1, es, ed):
  return pl.pallas_call(
      _tc4_body,
      out_shape=[jax.ShapeDtypeStruct((N, H), jnp.float32),
                 jax.ShapeDtypeStruct((N, H), jnp.float32)],
  )(dn0, dn1, es, ed)


def _tc5_body(p0_ref, p1_ref, hg0_ref, hg1_ref, hg2_ref, hg3_ref,
              ees_ref, rden_ref, bg_ref, node_ref, graph_ref):
  i = pl.program_id(0)
  hg_refs = (hg0_ref, hg1_ref, hg2_ref, hg3_ref)
  t = p0_ref[...] + p1_ref[...] + bg_ref[...]
  w = ees_ref[...] * rden_ref[...]
  for h in range(H):
    t = t + w[:, h:h + 1] * hg_refs[h][...]
  node = jnp.maximum(t, 0.0)
  node_ref[...] = node
  psum = jnp.sum(node, axis=0, keepdims=True)
  pmax = jnp.max(node, axis=0, keepdims=True)

  @pl.when(i == 0)
  def _():
    graph_ref[...] = jnp.concatenate([psum, pmax], axis=1)

  @pl.when(i > 0)
  def _():
    prev = graph_ref[...]
    graph_ref[...] = jnp.concatenate(
        [prev[:, :D] + psum, jnp.maximum(prev[:, D:], pmax)], axis=1)

  @pl.when(i == NG - 1)
  def _():
    cur = graph_ref[...]
    graph_ref[...] = jnp.concatenate(
        [cur[:, :D] * (1.0 / N), cur[:, D:]], axis=1)


def _tc5(p0, p1, hg0, hg1, hg2, hg3, ees, rden, bg_row):
  return pl.pallas_call(
      _tc5_body,
      grid=(NG,),
      in_specs=[_row_spec(D)] * 6 + [_row_spec(H), _row_spec(H),
                                     _full_spec(1, D)],
      out_specs=[_row_spec(D), _full_spec(1, 2 * D)],
      out_shape=[jax.ShapeDtypeStruct((N, D), jnp.float32),
                 jax.ShapeDtypeStruct((1, 2 * D), jnp.float32)],
  )(p0, p1, hg0, hg1, hg2, hg3, ees, rden, bg_row)


def kernel(x, edge_index, W0, b0, g0, be0, W1, b1, g1, be1, Wg,
           att_src, att_dst, bg):
  src = edge_index[0]
  dst = edge_index[1]

  degp = _sc_deg(dst)
  dp0 = degp[:N, 0:1]
  dp1 = degp[N2:N2 + N, 0:1]

  def _halves(a):
    return a[:, :D // 2], a[:, D // 2:]

  def _assemble(out):
    b = NC * N2
    p0 = jnp.concatenate([out[0:N], out[b:b + N]], axis=1)
    p1 = jnp.concatenate([out[N2:N2 + N], out[b + N2:b + N2 + N]], axis=1)
    return p0, p1

  hs0 = _tc1(x, W0, dp0, dp1)
  hs0a, hs0b = _halves(hs0)
  a0, a1 = _assemble(_sc_gcn(hs0a, hs0b, src, dst))
  hs1 = _tc2(a0, a1, hs0, dp0, dp1,
             b0.reshape(1, D), g0.reshape(1, D), be0.reshape(1, D), W1)
  hs1a, hs1b = _halves(hs1)
  g0p, g1p = _assemble(_sc_gcn(hs1a, hs1b, src, dst))

  a_src_mat = jnp.zeros((H * D, H), jnp.float32)
  a_dst_mat = jnp.zeros((H * D, H), jnp.float32)
  for h in range(H):
    a_src_mat = a_src_mat.at[h * D:(h + 1) * D, h].set(att_src[h])
    a_dst_mat = a_dst_mat.at[h * D:(h + 1) * D, h].set(att_dst[h])

  hg0, hg1, hg2, hg3, es, ed = _tc3(
      g0p, g1p, hs1, dp0, dp1,
      b1.reshape(1, D), g1.reshape(1, D), be1.reshape(1, D),
      Wg, a_src_mat, a_dst_mat)

  est_t = es.T.reshape(-1)
  edt_t = ed.T.reshape(-1)
  denp = _sc_den(src, dst, est_t, edt_t)
  rden, ees = _tc4(denp[:N, :H], denp[N2:N2 + N, :H], es, ed)

  hgsplit = [h for hg in (hg0, hg1, hg2, hg3) for h in _halves(hg)]
  hgas = [hgsplit[0], hgsplit[2], hgsplit[4], hgsplit[6]]
  hgbs = [hgsplit[1], hgsplit[3], hgsplit[5], hgsplit[7]]
  q0, q1 = _assemble(_sc_gat(src, dst, est_t, edt_t,
                             rden.T.reshape(-1), *hgas, *hgbs))
  node_emb, graph_emb = _tc5(q0, q1, hg0, hg1, hg2, hg3,
                             ees, rden, bg.reshape(1, D))
  return (graph_emb, node_emb)


# speedup vs baseline: 15.1350x; 1.5564x over previous
import functools

import jax
import jax.numpy as jnp
from jax import lax
from jax.experimental import pallas as pl
from jax.experimental.pallas import tpu as pltpu
from jax.experimental.pallas import tpu_sc as plsc

N = 10000
E = 320000
D = 128
H = 4

NC = 2
NS = 16
NW = NC * NS
EPW = E // NW
N2 = 10112
RPTW = N2 // NS
K = 400
NCHUNK = EPW // K

_mesh = plsc.VectorSubcoreMesh(
    core_axis_name="c", subcore_axis_name="s", num_cores=NC, num_subcores=NS)
_sc_params = pltpu.CompilerParams(
    use_tc_tiling_on_sc=False, needs_layout_passes=False)


def _fill_zero_2d(ref, nrows, ncols):
  zero16 = jnp.zeros((16,), jnp.float32)
  cpr = ncols // 16

  def body(t, _):
    i = t // cpr
    c = t % cpr
    ref[i, pl.ds(c * 16, 16)] = zero16
    return 0

  lax.fori_loop(0, nrows * cpr, body, 0)


def _edge_base(cid, sid):
  return (cid * NS + sid) * EPW


@functools.partial(
    pl.kernel,
    out_type=jax.ShapeDtypeStruct((NC * N2, 16), jnp.float32),
    mesh=_mesh,
    compiler_params=_sc_params,
    scratch_types=[
        pltpu.VMEM((K,), jnp.int32),
        pltpu.VMEM((K, 16), jnp.float32),
        pltpu.VMEM((RPTW, 16), jnp.float32),
        pltpu.VMEM_SHARED((N2, 16), jnp.float32),
    ],
)
def _sc_deg(dst_hbm, out_hbm, didx, onesv, zb, acc_sh):
  cid = lax.axis_index("c")
  sid = lax.axis_index("s")
  base = _edge_base(cid, sid)

  _fill_zero_2d(zb, RPTW, 16)
  pltpu.sync_copy(zb, acc_sh.at[pl.ds(sid * RPTW, RPTW)])

  one_row = (lax.iota(jnp.int32, 16) == 0).astype(jnp.float32)

  def fillones(i, _):
    onesv[i, pl.ds(0, 16)] = one_row
    return 0

  lax.fori_loop(0, K, fillones, 0)
  plsc.subcore_barrier()

  def chunk(i, _):
    pltpu.sync_copy(dst_hbm.at[pl.ds(base + i * K, K)], didx)
    pltpu.sync_copy(onesv, acc_sh.at[didx], add=True)
    return 0

  lax.fori_loop(0, NCHUNK, chunk, 0)
  plsc.subcore_barrier()
  pltpu.sync_copy(acc_sh.at[pl.ds(sid * RPTW, RPTW)],
                  out_hbm.at[pl.ds(cid * N2 + sid * RPTW, RPTW)])


@functools.partial(
    pl.kernel,
    out_type=jax.ShapeDtypeStruct((2 * NC * N2, D // 2), jnp.float32),
    mesh=_mesh,
    compiler_params=_sc_params,
    scratch_types=[
        pltpu.VMEM((K,), jnp.int32),
        pltpu.VMEM((K,), jnp.int32),
        pltpu.VMEM((K, D // 2), jnp.float32),
        pltpu.VMEM((120, D // 2), jnp.float32),
        pltpu.VMEM_SHARED((N2, D // 2), jnp.float32),
    ],
)
def _sc_gcn(hsa_hbm, hsb_hbm, src_hbm, dst_hbm, out_hbm,
            sidx, didx, rows, zb, acc_sh):
  cid = lax.axis_index("c")
  sid = lax.axis_index("s")
  base = _edge_base(cid, sid)
  hcols = (hsa_hbm, hsb_hbm)
  _fill_zero_2d(zb, 120, D // 2)

  for c in range(2):
    for j in range(5):
      pltpu.sync_copy(zb, acc_sh.at[pl.ds(sid * RPTW + j * 120, 120)])
    pltpu.sync_copy(zb.at[pl.ds(0, 32)],
                    acc_sh.at[pl.ds(sid * RPTW + 600, 32)])
    plsc.subcore_barrier()

    def chunk(i, _, c=c):
      pltpu.sync_copy(src_hbm.at[pl.ds(base + i * K, K)], sidx)
      pltpu.sync_copy(hcols[c].at[sidx], rows)
      pltpu.sync_copy(dst_hbm.at[pl.ds(base + i * K, K)], didx)
      pltpu.sync_copy(rows, acc_sh.at[didx], add=True)
      return 0

    lax.fori_loop(0, NCHUNK, chunk, 0)
    plsc.subcore_barrier()
    pltpu.sync_copy(
        acc_sh.at[pl.ds(sid * RPTW, RPTW)],
        out_hbm.at[pl.ds(c * NC * N2 + cid * N2 + sid * RPTW, RPTW)])


@functools.partial(
    pl.kernel,
    out_type=jax.ShapeDtypeStruct((NC * N2, 16), jnp.float32),
    mesh=_mesh,
    compiler_params=_sc_params,
    scratch_types=[
        pltpu.VMEM((K,), jnp.int32),
        pltpu.VMEM((K,), jnp.int32),
        pltpu.VMEM((K, 16), jnp.float32),
        pltpu.VMEM((H * N,), jnp.float32),
        pltpu.VMEM((H * N,), jnp.float32),
        pltpu.VMEM((RPTW, 16), jnp.float32),
        pltpu.VMEM_SHARED((N2, 16), jnp.float32),
    ],
)
def _sc_den(src_hbm, dst_hbm, est_hbm, edt_hbm, out_hbm,
            sidx, didx, eebuf, est_v, edt_v, zb, acc_sh):
  cid = lax.axis_index("c")
  sid = lax.axis_index("s")
  base = _edge_base(cid, sid)

  _fill_zero_2d(zb, RPTW, 16)
  pltpu.sync_copy(zb, acc_sh.at[pl.ds(sid * RPTW, RPTW)])
  _fill_zero_2d(eebuf, K, 16)
  pltpu.sync_copy(est_hbm, est_v)
  pltpu.sync_copy(edt_hbm, edt_v)
  plsc.subcore_barrier()

  hconst = [jnp.full((16,), h, jnp.int32) for h in range(H)]

  def chunk(i, _):
    pltpu.sync_copy(src_hbm.at[pl.ds(base + i * K, K)], sidx)
    pltpu.sync_copy(dst_hbm.at[pl.ds(base + i * K, K)], didx)
    for g in range(K // 16):
      s16 = sidx[pl.ds(g * 16, 16)]
      d16 = didx[pl.ds(g * 16, 16)]
      rowsid = jnp.arange(g * 16, g * 16 + 16, dtype=jnp.int32)
      for h in range(H):
        a = plsc.load_gather(est_v, [s16 + h * N])
        b = plsc.load_gather(edt_v, [d16 + h * N])
        s = a + b
        e = jnp.where(s >= 0, s, 0.2 * s)
        ee = jnp.exp(e)
        plsc.store_scatter(eebuf, [rowsid, hconst[h]], ee)
    pltpu.sync_copy(eebuf, acc_sh.at[didx], add=True)
    return 0

  lax.fori_loop(0, NCHUNK, chunk, 0)
  plsc.subcore_barrier()
  pltpu.sync_copy(acc_sh.at[pl.ds(sid * RPTW, RPTW)],
                  out_hbm.at[pl.ds(cid * N2 + sid * RPTW, RPTW)])


@functools.partial(
    pl.kernel,
    out_type=jax.ShapeDtypeStruct((2 * NC * N2, D // 2), jnp.float32),
    mesh=_mesh,
    compiler_params=_sc_params,
    scratch_types=[
        pltpu.VMEM((K,), jnp.int32),
        pltpu.VMEM((K,), jnp.int32),
        pltpu.VMEM((K, D // 2), jnp.float32),
        pltpu.VMEM((K,), jnp.float32),
        pltpu.VMEM((N,), jnp.float32),
        pltpu.VMEM((N,), jnp.float32),
        pltpu.VMEM((N,), jnp.float32),
        pltpu.VMEM((120, D // 2), jnp.float32),
        pltpu.VMEM_SHARED((N2, D // 2), jnp.float32),
    ],
)
def _sc_gat(src_hbm, dst_hbm, est_hbm, edt_hbm, rdent_hbm,
            hga0, hga1, hga2, hga3, hgb0, hgb1, hgb2, hgb3, out_hbm,
            sidx, didx, rows, abuf, esh_v, edh_v, rdh_v, zb, acc_sh):
  cid = lax.axis_index("c")
  sid = lax.axis_index("s")
  base = _edge_base(cid, sid)
  hgs = ((hga0, hga1, hga2, hga3), (hgb0, hgb1, hgb2, hgb3))
  _fill_zero_2d(zb, 120, D // 2)

  for c in range(2):
    for j in range(5):
      pltpu.sync_copy(zb, acc_sh.at[pl.ds(sid * RPTW + j * 120, 120)])
    pltpu.sync_copy(zb.at[pl.ds(0, 32)],
                    acc_sh.at[pl.ds(sid * RPTW + 600, 32)])
    plsc.subcore_barrier()

    for h in range(H):
      pltpu.sync_copy(est_hbm.at[pl.ds(h * N, N)], esh_v)
      pltpu.sync_copy(edt_hbm.at[pl.ds(h * N, N)], edh_v)
      pltpu.sync_copy(rdent_hbm.at[pl.ds(h * N, N)], rdh_v)

      def chunk(i, _, c=c, h=h):
        pltpu.sync_copy(src_hbm.at[pl.ds(base + i * K, K)], sidx)
        pltpu.sync_copy(dst_hbm.at[pl.ds(base + i * K, K)], didx)
        pltpu.sync_copy(hgs[c][h].at[sidx], rows)
        for g in range(K // 16):
          s16 = sidx[pl.ds(g * 16, 16)]
          d16 = didx[pl.ds(g * 16, 16)]
          a = plsc.load_gather(esh_v, [s16])
          b = plsc.load_gather(edh_v, [d16])
          s = a + b
          e = jnp.where(s >= 0, s, 0.2 * s)
          ee = jnp.exp(e)
          r = plsc.load_gather(rdh_v, [d16])
          abuf[pl.ds(g * 16, 16)] = ee * r

        def edge_body(k, _):
          k16 = jnp.full((16,), 0, jnp.int32) + k
          spl = plsc.load_gather(abuf, [k16])
          for cc in range(D // 32):
            cidx = jnp.arange(cc * 16, cc * 16 + 16, dtype=jnp.int32)
            v = plsc.load_gather(rows, [k16, cidx])
            plsc.store_scatter(rows, [k16, cidx], v * spl)
          return 0

        lax.fori_loop(0, K, edge_body, 0)
        pltpu.sync_copy(rows, acc_sh.at[didx], add=True)
        return 0

      lax.fori_loop(0, NCHUNK, chunk, 0)

    plsc.subcore_barrier()
    pltpu.sync_copy(
        acc_sh.at[pl.ds(sid * RPTW, RPTW)],
        out_hbm.at[pl.ds(c * NC * N2 + cid * N2 + sid * RPTW, RPTW)])


RB = 2000
NG = N // RB

_row_spec = lambda cols: pl.BlockSpec((RB, cols), lambda i: (i, 0))
_full_spec = lambda r, c: pl.BlockSpec((r, c), lambda i: (0, 0))


def _dinv_of(dp0, dp1):
  return 1.0 / jnp.sqrt(dp0 + dp1 + 1.0)


def _tc1_body(x_ref, w_ref, dp0_ref, dp1_ref, hs_ref):
  dinv = _dinv_of(dp0_ref[...], dp1_ref[...])
  h = jnp.dot(x_ref[...], w_ref[...], preferred_element_type=jnp.float32, precision=lax.Precision.HIGHEST)
  hs_ref[...] = h * dinv


def _tc1(x, w, dp0, dp1):
  return pl.pallas_call(
      _tc1_body,
      grid=(NG,),
      in_specs=[_row_spec(D), _full_spec(D, D), _row_spec(1), _row_spec(1)],
      out_specs=_row_spec(D),
      out_shape=jax.ShapeDtypeStruct((N, D), jnp.float32),
  )(x, w, dp0, dp1)


def _ln_relu(t, g, be):
  m = jnp.mean(t, axis=-1, keepdims=True)
  v = jnp.mean((t - m) * (t - m), axis=-1, keepdims=True)
  t = (t - m) / jnp.sqrt(v + 1e-5) * g + be
  return jnp.maximum(t, 0.0)


def _tc2_body(p0_ref, p1_ref, hs_ref, dp0_ref, dp1_ref, b_ref, g_ref,
              be_ref, w_ref, out_ref):
  dinv = _dinv_of(dp0_ref[...], dp1_ref[...])
  t = dinv * (p0_ref[...] + p1_ref[...] + hs_ref[...]) + b_ref[...]
  t = _ln_relu(t, g_ref[...], be_ref[...])
  out_ref[...] = jnp.dot(
      t, w_ref[...], preferred_element_type=jnp.float32, precision=lax.Precision.HIGHEST) * dinv


def _tc2(p0, p1, hs, dp0, dp1, b, g, be, w):
  return pl.pallas_call(
      _tc2_body,
      grid=(NG,),
      in_specs=[_row_spec(D), _row_spec(D), _row_spec(D), _row_spec(1),
                _row_spec(1), _full_spec(1, D), _full_spec(1, D),
                _full_spec(1, D), _full_spec(D, D)],
      out_specs=_row_spec(D),
      out_shape=jax.ShapeDtypeStruct((N, D), jnp.float32),
  )(p0, p1, hs, dp0, dp1, b, g, be, w)


def _tc3_body(p0_ref, p1_ref, hs_ref, dp0_ref, dp1_ref, b_ref, g_ref,
              be_ref, wg_ref, asrc_ref, adst_ref,
              hg0_ref, hg1_ref, hg2_ref, hg3_ref, es_ref, ed_ref):
  dinv = _dinv_of(dp0_ref[...], dp1_ref[...])
  t = dinv * (p0_ref[...] + p1_ref[...] + hs_ref[...]) + b_ref[...]
  t = _ln_relu(t, g_ref[...], be_ref[...])
  hg = jnp.dot(t, wg_ref[...], preferred_element_type=jnp.float32, precision=lax.Precision.HIGHEST)
  hg_refs = (hg0_ref, hg1_ref, hg2_ref, hg3_ref)
  for h in range(H):
    hg_refs[h][...] = hg[:, h * D:(h + 1) * D]
  es = jnp.dot(hg, asrc_ref[...], preferred_element_type=jnp.float32, precision=lax.Precision.HIGHEST)
  ed = jnp.dot(hg, adst_ref[...], preferred_element_type=jnp.float32, precision=lax.Precision.HIGHEST)
  es_ref[...] = es
  ed_ref[...] = ed


def _tc3(p0, p1, hs, dp0, dp1, b, g, be, wg, a_src_mat, a_dst_mat):
  return pl.pallas_call(
      _tc3_body,
      grid=(NG,),
      in_specs=[_row_spec(D), _row_spec(D), _row_spec(D), _row_spec(1),
                _row_spec(1), _full_spec(1, D), _full_spec(1, D),
                _full_spec(1, D), _full_spec(D, H * D),
                _full_spec(H * D, H), _full_spec(H * D, H)],
      out_specs=[_row_spec(D), _row_spec(D), _row_spec(D), _row_spec(D),
                 _row_spec(H), _row_spec(H)],
      out_shape=[jax.ShapeDtypeStruct((N, D), jnp.float32)] * 4 + [
          jax.ShapeDtypeStruct((N, H), jnp.float32),
          jax.ShapeDtypeStruct((N, H), jnp.float32),
      ],
  )(p0, p1, hs, dp0, dp1, b, g, be, wg, a_src_mat, a_dst_mat)


def _tc4_body(dn0_ref, dn1_ref, es_ref, ed_ref, rden_ref, ees_ref):
  s = es_ref[...] + ed_ref[...]
  e = jnp.where(s >= 0, s, 0.2 * s)
  ee = jnp.exp(e)
  den = dn0_ref[...] + dn1_ref[...] + ee
  rden_ref[...] = 0.25 / (den + 1e-30)
  ees_ref[...] = ee


def _tc4(dn0, dn1, es, ed):
  return pl.pallas_call(
      _tc4_body,
      out_shape=[jax.ShapeDtypeStruct((N, H), jnp.float32),
                 jax.ShapeDtypeStruct((N, H), jnp.float32)],
  )(dn0, dn1, es, ed)


def _tc5_body(p0_ref, p1_ref, hg0_ref, hg1_ref, hg2_ref, hg3_ref,
              ees_ref, rden_ref, bg_ref, node_ref, graph_ref):
  i = pl.program_id(0)
  hg_refs = (hg0_ref, hg1_ref, hg2_ref, hg3_ref)
  t = p0_ref[...] + p1_ref[...] + bg_ref[...]
  w = ees_ref[...] * rden_ref[...]
  for h in range(H):
    t = t + w[:, h:h + 1] * hg_refs[h][...]
  node = jnp.maximum(t, 0.0)
  node_ref[...] = node
  psum = jnp.sum(node, axis=0, keepdims=True)
  pmax = jnp.max(node, axis=0, keepdims=True)

  @pl.when(i == 0)
  def _():
    graph_ref[...] = jnp.concatenate([psum, pmax], axis=1)

  @pl.when(i > 0)
  def _():
    prev = graph_ref[...]
    graph_ref[...] = jnp.concatenate(
        [prev[:, :D] + psum, jnp.maximum(prev[:, D:], pmax)], axis=1)

  @pl.when(i == NG - 1)
  def _():
    cur = graph_ref[...]
    graph_ref[...] = jnp.concatenate(
        [cur[:, :D] * (1.0 / N), cur[:, D:]], axis=1)


def _tc5(p0, p1, hg0, hg1, hg2, hg3, ees, rden, bg_row):
  return pl.pallas_call(
      _tc5_body,
      grid=(NG,),
      in_specs=[_row_spec(D)] * 6 + [_row_spec(H), _row_spec(H),
                                     _full_spec(1, D)],
      out_specs=[_row_spec(D), _full_spec(1, 2 * D)],
      out_shape=[jax.ShapeDtypeStruct((N, D), jnp.float32),
                 jax.ShapeDtypeStruct((1, 2 * D), jnp.float32)],
  )(p0, p1, hg0, hg1, hg2, hg3, ees, rden, bg_row)


def kernel(x, edge_index, W0, b0, g0, be0, W1, b1, g1, be1, Wg,
           att_src, att_dst, bg):
  src = edge_index[0]
  dst = edge_index[1]

  degp = _sc_deg(dst)
  dp0 = degp[:N, 0:1]
  dp1 = degp[N2:N2 + N, 0:1]

  def _halves(a):
    return a[:, :D // 2], a[:, D // 2:]

  def _assemble(out):
    b = NC * N2
    p0 = jnp.concatenate([out[0:N], out[b:b + N]], axis=1)
    p1 = jnp.concatenate([out[N2:N2 + N], out[b + N2:b + N2 + N]], axis=1)
    return p0, p1

  hs0 = _tc1(x, W0, dp0, dp1)
  hs0a, hs0b = _halves(hs0)
  a0, a1 = _assemble(_sc_gcn(hs0a, hs0b, src, dst))
  hs1 = _tc2(a0, a1, hs0, dp0, dp1,
             b0.reshape(1, D), g0.reshape(1, D), be0.reshape(1, D), W1)
  hs1a, hs1b = _halves(hs1)
  g0p, g1p = _assemble(_sc_gcn(hs1a, hs1b, src, dst))

  a_src_mat = jnp.zeros((H * D, H), jnp.float32)
  a_dst_mat = jnp.zeros((H * D, H), jnp.float32)
  for h in range(H):
    a_src_mat = a_src_mat.at[h * D:(h + 1) * D, h].set(att_src[h])
    a_dst_mat = a_dst_mat.at[h * D:(h + 1) * D, h].set(att_dst[h])

  hg0, hg1, hg2, hg3, es, ed = _tc3(
      g0p, g1p, hs1, dp0, dp1,
      b1.reshape(1, D), g1.reshape(1, D), be1.reshape(1, D),
      Wg, a_src_mat, a_dst_mat)

  est_t = es.T.reshape(-1)
  edt_t = ed.T.reshape(-1)
  denp = _sc_den(src, dst, est_t, edt_t)
  rden, ees = _tc4(denp[:N, :H], denp[N2:N2 + N, :H], es, ed)

  hgsplit = [h for hg in (hg0, hg1, hg2, hg3) for h in _halves(hg)]
  hgas = [hgsplit[0], hgsplit[2], hgsplit[4], hgsplit[6]]
  hgbs = [hgsplit[1], hgsplit[3], hgsplit[5], hgsplit[7]]
  q0, q1 = _assemble(_sc_gat(src, dst, est_t, edt_t,
                             rden.T.reshape(-1), *hgas, *hgbs))
  node_emb, graph_emb = _tc5(q0, q1, hg0, hg1, hg2, hg3,
                             ees, rden, bg.reshape(1, D))
  return (graph_emb, node_emb)
